# Initial kernel scaffold; baseline (speedup 1.0000x reference)
import jax
import jax.numpy as jnp
from jax.experimental import pallas as pl


def kernel(x, ctx, E, W):
    raise NotImplementedError("write your pallas kernel here")


# speedup vs baseline: 21.8646x; 21.8646x over previous
import numpy as np
import jax
import jax.numpy as jnp
from jax import lax
from jax.experimental import pallas as pl
from jax.experimental.pallas import tpu as pltpu

B = 32
K = 4
V = 100000
ALPHA = 0.6
END = 2
L = 8
D = 64
NEG = -1e9
PADV = -1e30
R = B * K
CW = 4096
NC = 25
VP = NC * CW
BW = 512
NBC = CW // BW
NB = NC * NBC
NBP = 256
NSTEP = L - 1
BIGI = 2 ** 30
HI = jax.lax.Precision.HIGHEST


def _f32(x):
    return x.astype(jnp.float32)


def _body(x_ref, ctx_ref, pen_ref, e_hbm, w_ref,
          out_ref, score_ref,
          tgt_s, out_s, logp_s, lpr_s, score_s, flag_s,
          h_s, mb_s, lsc, cand_s, stage_s,
          ptok_s, stage_sm, tok_sm, sem):
    i = pl.program_id(0)
    c = pl.program_id(1)
    step = i + 1

    @pl.when(c == 0)
    def _prologue():
        @pl.when(i == 0)
        def _init():
            xv = x_ref[...]
            t0 = jnp.concatenate([xv] * K, axis=1)
            tgt_s[...] = t0
            out_s[...] = t0
            kl = lax.broadcasted_iota(jnp.int32, (B, K), 1)
            logp_s[...] = jnp.where(kl == 0, 0.0, NEG).astype(jnp.float32)
            rl = lax.broadcasted_iota(jnp.int32, (R, 1), 0)
            lpr_s[...] = jnp.where(rl % K == 0, 0.0, NEG).astype(jnp.float32)
            score_s[...] = jnp.full((B, K), NEG, jnp.float32)
            flag_s[...] = jnp.zeros((B, K), jnp.float32)

        mb_s[...] = jnp.full((R, NBP), PADV, jnp.float32)

        lanes = lax.broadcasted_iota(jnp.int32, (B, K * L), 1)
        pmask = (lanes % L) == (step - 1)
        tg = tgt_s[...]
        masked = jnp.where(pmask, tg, 0)
        gm = (lax.broadcasted_iota(jnp.int32, (K * L, K), 0) // L
              == lax.broadcasted_iota(jnp.int32, (K * L, K), 1))
        prev = jnp.dot(_f32(masked), _f32(gm), precision=HI,
                       preferred_element_type=jnp.float32)
        ptok_s[...] = prev.astype(jnp.int32)

        cp = pltpu.make_async_copy(ptok_s, tok_sm, sem)
        cp.start()
        cp.wait()

        def _issue(r, carry):
            b = r // K
            k = r - b * K
            t = tok_sm[b, k]
            pltpu.make_async_copy(e_hbm.at[pl.ds(t, 1), :],
                                  h_s.at[pl.ds(r, 1), :], sem).start()
            return carry
        lax.fori_loop(0, R, _issue, 0)

        def _drain(r, carry):
            pltpu.make_async_copy(e_hbm.at[pl.ds(0, 1), :],
                                  h_s.at[pl.ds(r, 1), :], sem).wait()
            return carry
        lax.fori_loop(0, R, _drain, 0)

        cx = ctx_ref[:, pl.ds(step, 1), :].reshape(B, D)
        sm = (lax.broadcasted_iota(jnp.int32, (R, B), 0) // K
              == lax.broadcasted_iota(jnp.int32, (R, B), 1))
        hb = jnp.dot(_f32(sm), cx, precision=HI,
                     preferred_element_type=jnp.float32)
        h_s[...] = h_s[...] + hb

    w = w_ref[...]
    h = h_s[...]
    logits = jnp.dot(h, w, preferred_element_type=jnp.float32)
    lanei = lax.broadcasted_iota(jnp.int32, (R, CW), 1)
    lc = jnp.where(lanei + c * CW < V, logits, PADV)

    bms = []
    mb = mb_s[...]
    lane_nbp = lax.broadcasted_iota(jnp.int32, (R, NBP), 1)
    for g in range(NBC):
        lsc[c, g] = lc[:, g * BW:(g + 1) * BW]
        f = jnp.maximum(
            jnp.maximum(lc[:, g * BW:g * BW + 128],
                        lc[:, g * BW + 128:g * BW + 256]),
            jnp.maximum(lc[:, g * BW + 256:g * BW + 384],
                        lc[:, g * BW + 384:g * BW + 512]))
        bm = jnp.max(f, axis=1, keepdims=True)
        mb = jnp.where(lane_nbp == c * NBC + g, bm, mb)
        bms.append(bm)
    mb_s[...] = mb

    @pl.when(c == NC - 1)
    def _merge():
        m = jnp.max(mb_s[...], axis=1, keepdims=True)

        def _sumexp(c2, acc):
            for g in range(NBC):
                e = jnp.exp(lsc[c2, g][...] - m)
                acc = acc + ((e[:, 0:128] + e[:, 128:256])
                             + (e[:, 256:384] + e[:, 384:512]))
            return acc
        svec = lax.fori_loop(0, NC, _sumexp,
                             jnp.zeros((R, 128), jnp.float32))
        logsum = jnp.log(jnp.sum(svec, axis=1, keepdims=True))
        lpr = lpr_s[...]
        ymb = lpr + ((mb_s[...] - m) - logsum)

        ym4 = jnp.zeros((B, K * NBP), jnp.float32)
        lpk = jnp.zeros((B, K), jnp.float32)
        mk = jnp.zeros((B, K), jnp.float32)
        lsk = jnp.zeros((B, K), jnp.float32)
        lkiota = lax.broadcasted_iota(jnp.int32, (B, K), 1)
        for k in range(K):
            sk = (lax.broadcasted_iota(jnp.int32, (B, R), 1)
                  == K * lax.broadcasted_iota(jnp.int32, (B, R), 0) + k)
            pk = (lax.broadcasted_iota(jnp.int32, (NBP, K * NBP), 1)
                  == lax.broadcasted_iota(jnp.int32, (NBP, K * NBP), 0)
                  + NBP * k)
            skf = _f32(sk)
            yk = jnp.dot(skf, ymb, precision=HI,
                         preferred_element_type=jnp.float32)
            ym4 = ym4 + jnp.dot(yk, _f32(pk), precision=HI,
                                preferred_element_type=jnp.float32)
            lpk = jnp.where(lkiota == k,
                            jnp.dot(skf, lpr, precision=HI,
                                    preferred_element_type=jnp.float32), lpk)
            mk = jnp.where(lkiota == k,
                           jnp.dot(skf, m, precision=HI,
                                   preferred_element_type=jnp.float32), mk)
            lsk = jnp.where(lkiota == k,
                            jnp.dot(skf, logsum, precision=HI,
                                    preferred_element_type=jnp.float32), lsk)

        lane4 = lax.broadcasted_iota(jnp.int32, (B, K * NBP), 1)
        wk = ym4
        sels = []
        for _t in range(8):
            v = jnp.max(wk, axis=1, keepdims=True)
            a = jnp.min(jnp.where(wk == v, lane4, BIGI),
                        axis=1, keepdims=True)
            wk = jnp.where(lane4 == a, PADV, wk)
            sels.append(a)
        asel = jnp.concatenate(sels, axis=1)
        ksel = asel // NBP
        bsel = asel - ksel * NBP
        csel = bsel // NBC
        gsel = bsel - csel * NBC
        biota = lax.broadcasted_iota(jnp.int32, (B, 8), 0)
        srcrow = biota * K + ksel

        stage_s[...] = jnp.concatenate([srcrow, csel, gsel], axis=1)
        cps = pltpu.make_async_copy(stage_s, stage_sm, sem)
        cps.start()
        cps.wait()

        def _gather(b, carry):
            for t in range(8):
                r = stage_sm[b, t]
                cc = stage_sm[b, 8 + t]
                gg = stage_sm[b, 16 + t]
                cand_s[pl.ds(b, 1), t * BW:(t + 1) * BW] = (
                    lsc[cc, gg, pl.ds(r, 1), :])
            return carry
        lax.fori_loop(0, B, _gather, 0)

        iotab = lax.broadcasted_iota(jnp.int32, (B, BW), 1)
        ycand_parts = []
        flat_parts = []
        for t in range(8):
            kt = ksel[:, t:t + 1]
            base = (csel[:, t:t + 1] * CW + gsel[:, t:t + 1] * BW)
            km = lkiota == kt
            lp_t = jnp.sum(jnp.where(km, lpk, 0.0), axis=1, keepdims=True)
            m_t = jnp.sum(jnp.where(km, mk, 0.0), axis=1, keepdims=True)
            ls_t = jnp.sum(jnp.where(km, lsk, 0.0), axis=1, keepdims=True)
            raw = cand_s[:, t * BW:(t + 1) * BW]
            ycand_parts.append(lp_t + ((raw - m_t) - ls_t))
            flat_parts.append(kt * V + base + iotab)
        ycand = jnp.concatenate(ycand_parts, axis=1)
        flatm = jnp.concatenate(flat_parts, axis=1)

        wc = ycand
        lps, flats = [], []
        for _t in range(8):
            v = jnp.max(wc, axis=1, keepdims=True)
            af = jnp.min(jnp.where(wc == v, flatm, BIGI),
                         axis=1, keepdims=True)
            wc = jnp.where(flatm == af, PADV, wc)
            lps.append(v)
            flats.append(af)
        lp8 = jnp.concatenate(lps, axis=1)
        flat8 = jnp.concatenate(flats, axis=1)
        beam8 = flat8 // V
        tok8 = flat8 - beam8 * V

        fs = (tok8 == END).astype(jnp.float32)
        lp2 = lp8 + fs * NEG
        penm = (lax.broadcasted_iota(jnp.int32, (8, L), 1) == step)
        pen_rows = jnp.sum(jnp.where(penm, pen_ref[...], 0.0),
                           axis=1, keepdims=True)
        pen = jnp.max(pen_rows, axis=0, keepdims=True)
        sc8 = lp8 / pen + (1.0 - fs) * NEG

        lane8b = lax.broadcasted_iota(jnp.int32, (B, 8), 1)
        tg_old = tgt_s[...]
        poslane = (lax.broadcasted_iota(jnp.int32, (B, L), 1) == step)

        def topk4_and_rebuild(keys):
            wk2 = keys
            vals, seqs, fsel = [], [], []
            for _j in range(K):
                v = jnp.max(wk2, axis=1, keepdims=True)
                a = jnp.min(jnp.where(wk2 == v, lane8b, BIGI),
                            axis=1, keepdims=True)
                wk2 = jnp.where(lane8b == a, PADV, wk2)
                m8 = lane8b == a
                sbeam = jnp.sum(jnp.where(m8, beam8, 0), axis=1,
                                keepdims=True)
                stok = jnp.sum(jnp.where(m8, tok8, 0), axis=1, keepdims=True)
                sfs = jnp.sum(jnp.where(m8, fs, 0.0), axis=1, keepdims=True)
                seq = jnp.zeros((B, L), jnp.int32)
                for s in range(K):
                    seq = seq + jnp.where(sbeam == s,
                                          tg_old[:, s * L:(s + 1) * L], 0)
                seq = jnp.where(poslane, stok, seq)
                vals.append(v)
                seqs.append(seq)
                fsel.append(sfs)
            return (jnp.concatenate(vals, axis=1),
                    jnp.concatenate(seqs, axis=1),
                    jnp.concatenate(fsel, axis=1))

        logp_new, tgt_new, _ = topk4_and_rebuild(lp2)
        score_new, out_new, flag_new = topk4_and_rebuild(sc8)

        tgt_s[...] = tgt_new
        out_s[...] = out_new
        logp_s[...] = logp_new
        score_s[...] = score_new
        flag_s[...] = flag_new

        smr = (lax.broadcasted_iota(jnp.int32, (R, B), 0) // K
               == lax.broadcasted_iota(jnp.int32, (R, B), 1))
        arep = jnp.dot(_f32(smr), logp_new, precision=HI,
                       preferred_element_type=jnp.float32)
        selc = (lax.broadcasted_iota(jnp.int32, (R, K), 1)
                == lax.broadcasted_iota(jnp.int32, (R, K), 0) % K)
        lpr_s[...] = jnp.sum(jnp.where(selc, arep, 0.0),
                             axis=1, keepdims=True)

        @pl.when(i == NSTEP - 1)
        def _fin():
            done = jnp.max(flag_new, axis=1, keepdims=True)
            out_ref[...] = jnp.where(done > 0, out_new, tgt_new)
            score_ref[...] = jnp.where(done > 0, score_new, logp_new)


def kernel(x, ctx, E, W):
    wp = jnp.pad(W, ((0, 0), (0, VP - V)))
    pens = np.zeros((8, L), np.float32)
    for s in range(L):
        pens[:, s] = np.float32(((5.0 + (s + 1)) / 6.0) ** ALPHA)
    pens_j = jnp.asarray(pens)

    out2, score = pl.pallas_call(
        _body,
        grid=(NSTEP, NC),
        in_specs=[
            pl.BlockSpec((B, L), lambda i, c: (0, 0)),
            pl.BlockSpec((B, L, D), lambda i, c: (0, 0, 0)),
            pl.BlockSpec((8, L), lambda i, c: (0, 0)),
            pl.BlockSpec(memory_space=pltpu.MemorySpace.HBM),
            pl.BlockSpec((D, CW), lambda i, c: (0, c)),
        ],
        out_specs=[
            pl.BlockSpec((B, K * L), lambda i, c: (0, 0)),
            pl.BlockSpec((B, K), lambda i, c: (0, 0)),
        ],
        out_shape=[
            jax.ShapeDtypeStruct((B, K * L), jnp.int32),
            jax.ShapeDtypeStruct((B, K), jnp.float32),
        ],
        scratch_shapes=[
            pltpu.VMEM((B, K * L), jnp.int32),
            pltpu.VMEM((B, K * L), jnp.int32),
            pltpu.VMEM((B, K), jnp.float32),
            pltpu.VMEM((R, 1), jnp.float32),
            pltpu.VMEM((B, K), jnp.float32),
            pltpu.VMEM((B, K), jnp.float32),
            pltpu.VMEM((R, D), jnp.float32),
            pltpu.VMEM((R, NBP), jnp.float32),
            pltpu.VMEM((NC, NBC, R, BW), jnp.float32),
            pltpu.VMEM((B, 8 * BW), jnp.float32),
            pltpu.VMEM((B, 24), jnp.int32),
            pltpu.VMEM((B, K), jnp.int32),
            pltpu.SMEM((B, 24), jnp.int32),
            pltpu.SMEM((B, K), jnp.int32),
            pltpu.SemaphoreType.DMA,
        ],
        compiler_params=pltpu.CompilerParams(
            dimension_semantics=("arbitrary", "arbitrary")),
    )(x, ctx, pens_j, E, wp)
    return out2.reshape(B, K, L), score


# speedup vs baseline: 23.6770x; 1.0829x over previous
import numpy as np
import jax
import jax.numpy as jnp
from jax import lax
from jax.experimental import pallas as pl
from jax.experimental.pallas import tpu as pltpu

B = 32
K = 4
V = 100000
ALPHA = 0.6
END = 2
L = 8
D = 64
NEG = -1e9
PADV = -1e30
R = B * K
CW = 6144
NC = 17
VP = NC * CW
BW = 512
NBC = CW // BW
NB = NC * NBC
NBP = 256
NSTEP = L - 1
BIGI = 2 ** 30
HI = jax.lax.Precision.HIGHEST


def _f32(x):
    return x.astype(jnp.float32)


def _body(x_ref, ctx_ref, pen_ref, e_hbm, w_ref,
          out_ref, score_ref,
          tgt_s, out_s, logp_s, lpr_s, score_s, flag_s,
          h_s, mb_s, lsc, cand_s, stage_s,
          ptok_s, stage_sm, tok_sm, sem):
    i = pl.program_id(0)
    c = pl.program_id(1)
    step = i + 1

    @pl.when(c == 0)
    def _prologue():
        @pl.when(i == 0)
        def _init():
            xv = x_ref[...]
            t0 = jnp.concatenate([xv] * K, axis=1)
            tgt_s[...] = t0
            out_s[...] = t0
            kl = lax.broadcasted_iota(jnp.int32, (B, K), 1)
            logp_s[...] = jnp.where(kl == 0, 0.0, NEG).astype(jnp.float32)
            rl = lax.broadcasted_iota(jnp.int32, (R, 1), 0)
            lpr_s[...] = jnp.where(rl % K == 0, 0.0, NEG).astype(jnp.float32)
            score_s[...] = jnp.full((B, K), NEG, jnp.float32)
            flag_s[...] = jnp.zeros((B, K), jnp.float32)

        mb_s[...] = jnp.full((R, NBP), PADV, jnp.float32)

        lanes = lax.broadcasted_iota(jnp.int32, (B, K * L), 1)
        pmask = (lanes % L) == (step - 1)
        tg = tgt_s[...]
        masked = jnp.where(pmask, tg, 0)
        gm = (lax.broadcasted_iota(jnp.int32, (K * L, K), 0) // L
              == lax.broadcasted_iota(jnp.int32, (K * L, K), 1))
        prev = jnp.dot(_f32(masked), _f32(gm), precision=HI,
                       preferred_element_type=jnp.float32)
        ptok_s[...] = prev.astype(jnp.int32)

        cp = pltpu.make_async_copy(ptok_s, tok_sm, sem)
        cp.start()
        cp.wait()

        def _issue(r, carry):
            b = r // K
            k = r - b * K
            t = tok_sm[b, k]
            pltpu.make_async_copy(e_hbm.at[pl.ds(t, 1), :],
                                  h_s.at[pl.ds(r, 1), :], sem).start()
            return carry
        lax.fori_loop(0, R, _issue, 0)

        def _drain(r, carry):
            pltpu.make_async_copy(e_hbm.at[pl.ds(0, 1), :],
                                  h_s.at[pl.ds(r, 1), :], sem).wait()
            return carry
        lax.fori_loop(0, R, _drain, 0)

        cx = ctx_ref[:, pl.ds(step, 1), :].reshape(B, D)
        sm = (lax.broadcasted_iota(jnp.int32, (R, B), 0) // K
              == lax.broadcasted_iota(jnp.int32, (R, B), 1))
        hb = jnp.dot(_f32(sm), cx, precision=HI,
                     preferred_element_type=jnp.float32)
        h_s[...] = h_s[...] + hb

    w = w_ref[...]
    h = h_s[...]
    logits = jnp.dot(h, w, preferred_element_type=jnp.float32)
    lanei = lax.broadcasted_iota(jnp.int32, (R, CW), 1)
    lc = jnp.where(lanei + c * CW < V, logits, PADV)

    bms = []
    mb = mb_s[...]
    lane_nbp = lax.broadcasted_iota(jnp.int32, (R, NBP), 1)
    for g in range(NBC):
        lsc[c, g] = lc[:, g * BW:(g + 1) * BW]
        f = jnp.maximum(
            jnp.maximum(lc[:, g * BW:g * BW + 128],
                        lc[:, g * BW + 128:g * BW + 256]),
            jnp.maximum(lc[:, g * BW + 256:g * BW + 384],
                        lc[:, g * BW + 384:g * BW + 512]))
        bm = jnp.max(f, axis=1, keepdims=True)
        mb = jnp.where(lane_nbp == c * NBC + g, bm, mb)
        bms.append(bm)
    mb_s[...] = mb

    @pl.when(c == NC - 1)
    def _merge():
        m = jnp.max(mb_s[...], axis=1, keepdims=True)

        def _sumexp(c2, acc):
            for g in range(NBC):
                e = jnp.exp(lsc[c2, g][...] - m)
                acc = acc + ((e[:, 0:128] + e[:, 128:256])
                             + (e[:, 256:384] + e[:, 384:512]))
            return acc
        svec = lax.fori_loop(0, NC, _sumexp,
                             jnp.zeros((R, 128), jnp.float32))
        logsum = jnp.log(jnp.sum(svec, axis=1, keepdims=True))
        lpr = lpr_s[...]
        ymb = lpr + ((mb_s[...] - m) - logsum)

        ym4 = jnp.zeros((B, K * NBP), jnp.float32)
        lpk = jnp.zeros((B, K), jnp.float32)
        mk = jnp.zeros((B, K), jnp.float32)
        lsk = jnp.zeros((B, K), jnp.float32)
        lkiota = lax.broadcasted_iota(jnp.int32, (B, K), 1)
        for k in range(K):
            sk = (lax.broadcasted_iota(jnp.int32, (B, R), 1)
                  == K * lax.broadcasted_iota(jnp.int32, (B, R), 0) + k)
            pk = (lax.broadcasted_iota(jnp.int32, (NBP, K * NBP), 1)
                  == lax.broadcasted_iota(jnp.int32, (NBP, K * NBP), 0)
                  + NBP * k)
            skf = _f32(sk)
            yk = jnp.dot(skf, ymb, precision=HI,
                         preferred_element_type=jnp.float32)
            ym4 = ym4 + jnp.dot(yk, _f32(pk), precision=HI,
                                preferred_element_type=jnp.float32)
            lpk = jnp.where(lkiota == k,
                            jnp.dot(skf, lpr, precision=HI,
                                    preferred_element_type=jnp.float32), lpk)
            mk = jnp.where(lkiota == k,
                           jnp.dot(skf, m, precision=HI,
                                   preferred_element_type=jnp.float32), mk)
            lsk = jnp.where(lkiota == k,
                            jnp.dot(skf, logsum, precision=HI,
                                    preferred_element_type=jnp.float32), lsk)

        lane4 = lax.broadcasted_iota(jnp.int32, (B, K * NBP), 1)
        wk = ym4
        sels = []
        for _t in range(8):
            v = jnp.max(wk, axis=1, keepdims=True)
            a = jnp.min(jnp.where(wk == v, lane4, BIGI),
                        axis=1, keepdims=True)
            wk = jnp.where(lane4 == a, PADV, wk)
            sels.append(a)
        asel = jnp.concatenate(sels, axis=1)
        ksel = asel // NBP
        bsel = asel - ksel * NBP
        csel = bsel // NBC
        gsel = bsel - csel * NBC
        biota = lax.broadcasted_iota(jnp.int32, (B, 8), 0)
        srcrow = biota * K + ksel

        stage_s[...] = jnp.concatenate([srcrow, csel, gsel], axis=1)
        cps = pltpu.make_async_copy(stage_s, stage_sm, sem)
        cps.start()
        cps.wait()

        def _gather(b, carry):
            for t in range(8):
                r = stage_sm[b, t]
                cc = stage_sm[b, 8 + t]
                gg = stage_sm[b, 16 + t]
                cand_s[pl.ds(b, 1), t * BW:(t + 1) * BW] = (
                    lsc[cc, gg, pl.ds(r, 1), :])
            return carry
        lax.fori_loop(0, B, _gather, 0)

        iotab = lax.broadcasted_iota(jnp.int32, (B, BW), 1)
        ycand_parts = []
        flat_parts = []
        for t in range(8):
            kt = ksel[:, t:t + 1]
            base = (csel[:, t:t + 1] * CW + gsel[:, t:t + 1] * BW)
            km = lkiota == kt
            lp_t = jnp.sum(jnp.where(km, lpk, 0.0), axis=1, keepdims=True)
            m_t = jnp.sum(jnp.where(km, mk, 0.0), axis=1, keepdims=True)
            ls_t = jnp.sum(jnp.where(km, lsk, 0.0), axis=1, keepdims=True)
            raw = cand_s[:, t * BW:(t + 1) * BW]
            ycand_parts.append(lp_t + ((raw - m_t) - ls_t))
            flat_parts.append(kt * V + base + iotab)
        ycand = jnp.concatenate(ycand_parts, axis=1)
        flatm = jnp.concatenate(flat_parts, axis=1)

        wc = ycand
        lps, flats = [], []
        for _t in range(8):
            v = jnp.max(wc, axis=1, keepdims=True)
            af = jnp.min(jnp.where(wc == v, flatm, BIGI),
                         axis=1, keepdims=True)
            wc = jnp.where(flatm == af, PADV, wc)
            lps.append(v)
            flats.append(af)
        lp8 = jnp.concatenate(lps, axis=1)
        flat8 = jnp.concatenate(flats, axis=1)
        beam8 = flat8 // V
        tok8 = flat8 - beam8 * V

        fs = (tok8 == END).astype(jnp.float32)
        lp2 = lp8 + fs * NEG
        penm = (lax.broadcasted_iota(jnp.int32, (8, L), 1) == step)
        pen_rows = jnp.sum(jnp.where(penm, pen_ref[...], 0.0),
                           axis=1, keepdims=True)
        pen = jnp.max(pen_rows, axis=0, keepdims=True)
        sc8 = lp8 / pen + (1.0 - fs) * NEG

        lane8b = lax.broadcasted_iota(jnp.int32, (B, 8), 1)
        tg_old = tgt_s[...]
        poslane = (lax.broadcasted_iota(jnp.int32, (B, L), 1) == step)

        def topk4_and_rebuild(keys):
            wk2 = keys
            vals, seqs, fsel = [], [], []
            for _j in range(K):
                v = jnp.max(wk2, axis=1, keepdims=True)
                a = jnp.min(jnp.where(wk2 == v, lane8b, BIGI),
                            axis=1, keepdims=True)
                wk2 = jnp.where(lane8b == a, PADV, wk2)
                m8 = lane8b == a
                sbeam = jnp.sum(jnp.where(m8, beam8, 0), axis=1,
                                keepdims=True)
                stok = jnp.sum(jnp.where(m8, tok8, 0), axis=1, keepdims=True)
                sfs = jnp.sum(jnp.where(m8, fs, 0.0), axis=1, keepdims=True)
                seq = jnp.zeros((B, L), jnp.int32)
                for s in range(K):
                    seq = seq + jnp.where(sbeam == s,
                                          tg_old[:, s * L:(s + 1) * L], 0)
                seq = jnp.where(poslane, stok, seq)
                vals.append(v)
                seqs.append(seq)
                fsel.append(sfs)
            return (jnp.concatenate(vals, axis=1),
                    jnp.concatenate(seqs, axis=1),
                    jnp.concatenate(fsel, axis=1))

        logp_new, tgt_new, _ = topk4_and_rebuild(lp2)
        score_new, out_new, flag_new = topk4_and_rebuild(sc8)

        tgt_s[...] = tgt_new
        out_s[...] = out_new
        logp_s[...] = logp_new
        score_s[...] = score_new
        flag_s[...] = flag_new

        smr = (lax.broadcasted_iota(jnp.int32, (R, B), 0) // K
               == lax.broadcasted_iota(jnp.int32, (R, B), 1))
        arep = jnp.dot(_f32(smr), logp_new, precision=HI,
                       preferred_element_type=jnp.float32)
        selc = (lax.broadcasted_iota(jnp.int32, (R, K), 1)
                == lax.broadcasted_iota(jnp.int32, (R, K), 0) % K)
        lpr_s[...] = jnp.sum(jnp.where(selc, arep, 0.0),
                             axis=1, keepdims=True)

        @pl.when(i == NSTEP - 1)
        def _fin():
            done = jnp.max(flag_new, axis=1, keepdims=True)
            out_ref[...] = jnp.where(done > 0, out_new, tgt_new)
            score_ref[...] = jnp.where(done > 0, score_new, logp_new)


def kernel(x, ctx, E, W):
    wp = jnp.pad(W, ((0, 0), (0, VP - V)))
    pens = np.zeros((8, L), np.float32)
    for s in range(L):
        pens[:, s] = np.float32(((5.0 + (s + 1)) / 6.0) ** ALPHA)
    pens_j = jnp.asarray(pens)

    out2, score = pl.pallas_call(
        _body,
        grid=(NSTEP, NC),
        in_specs=[
            pl.BlockSpec((B, L), lambda i, c: (0, 0)),
            pl.BlockSpec((B, L, D), lambda i, c: (0, 0, 0)),
            pl.BlockSpec((8, L), lambda i, c: (0, 0)),
            pl.BlockSpec(memory_space=pltpu.MemorySpace.HBM),
            pl.BlockSpec((D, CW), lambda i, c: (0, c)),
        ],
        out_specs=[
            pl.BlockSpec((B, K * L), lambda i, c: (0, 0)),
            pl.BlockSpec((B, K), lambda i, c: (0, 0)),
        ],
        out_shape=[
            jax.ShapeDtypeStruct((B, K * L), jnp.int32),
            jax.ShapeDtypeStruct((B, K), jnp.float32),
        ],
        scratch_shapes=[
            pltpu.VMEM((B, K * L), jnp.int32),
            pltpu.VMEM((B, K * L), jnp.int32),
            pltpu.VMEM((B, K), jnp.float32),
            pltpu.VMEM((R, 1), jnp.float32),
            pltpu.VMEM((B, K), jnp.float32),
            pltpu.VMEM((B, K), jnp.float32),
            pltpu.VMEM((R, D), jnp.float32),
            pltpu.VMEM((R, NBP), jnp.float32),
            pltpu.VMEM((NC, NBC, R, BW), jnp.float32),
            pltpu.VMEM((B, 8 * BW), jnp.float32),
            pltpu.VMEM((B, 24), jnp.int32),
            pltpu.VMEM((B, K), jnp.int32),
            pltpu.SMEM((B, 24), jnp.int32),
            pltpu.SMEM((B, K), jnp.int32),
            pltpu.SemaphoreType.DMA,
        ],
        compiler_params=pltpu.CompilerParams(
            dimension_semantics=("arbitrary", "arbitrary")),
    )(x, ctx, pens_j, E, wp)
    return out2.reshape(B, K, L), score


# speedup vs baseline: 24.4230x; 1.0315x over previous
import numpy as np
import jax
import jax.numpy as jnp
from jax import lax
from jax.experimental import pallas as pl
from jax.experimental.pallas import tpu as pltpu

B = 32
K = 4
V = 100000
ALPHA = 0.6
END = 2
L = 8
D = 64
NEG = -1e9
PADV = -1e30
R = B * K
CW = 6144
NC = 17
VP = NC * CW
BW = 512
NBC = CW // BW
NB = NC * NBC
NBP = 256
NSTEP = L - 1
BIGI = 2 ** 30
HI = jax.lax.Precision.HIGHEST


def _f32(x):
    return x.astype(jnp.float32)


def _body(x_ref, ctx_ref, pen_ref, e_hbm, w_ref,
          out_ref, score_ref,
          tgt_s, out_s, logp_s, lpr_s, score_s, flag_s,
          h_s, mb_s, lsc, cand_s, stage_s,
          ptok_s, stage_sm, tok_sm, sem):
    i = pl.program_id(0)
    c = pl.program_id(1)
    step = i + 1

    @pl.when(c == 0)
    def _prologue():
        @pl.when(i == 0)
        def _init():
            xv = x_ref[...]
            t0 = jnp.concatenate([xv] * K, axis=1)
            tgt_s[...] = t0
            out_s[...] = t0
            kl = lax.broadcasted_iota(jnp.int32, (B, K), 1)
            logp_s[...] = jnp.where(kl == 0, 0.0, NEG).astype(jnp.float32)
            rl = lax.broadcasted_iota(jnp.int32, (R, 1), 0)
            lpr_s[...] = jnp.where(rl % K == 0, 0.0, NEG).astype(jnp.float32)
            score_s[...] = jnp.full((B, K), NEG, jnp.float32)
            flag_s[...] = jnp.zeros((B, K), jnp.float32)

        mb_s[...] = jnp.full((R, NBP), PADV, jnp.float32)

        lanes = lax.broadcasted_iota(jnp.int32, (B, K * L), 1)
        pmask = (lanes % L) == (step - 1)
        tg = tgt_s[...]
        masked = jnp.where(pmask, tg, 0)
        gm = (lax.broadcasted_iota(jnp.int32, (K * L, K), 0) // L
              == lax.broadcasted_iota(jnp.int32, (K * L, K), 1))
        prev = jnp.dot(_f32(masked), _f32(gm), precision=HI,
                       preferred_element_type=jnp.float32)
        ptok_s[...] = prev.astype(jnp.int32)

        cp = pltpu.make_async_copy(ptok_s, tok_sm, sem)
        cp.start()
        cp.wait()

        def _issue(r, carry):
            b = r // K
            k = r - b * K
            t = tok_sm[b, k]
            pltpu.make_async_copy(e_hbm.at[pl.ds(t, 1), :],
                                  h_s.at[pl.ds(r, 1), :], sem).start()
            return carry
        lax.fori_loop(0, R, _issue, 0)

        def _drain(r, carry):
            pltpu.make_async_copy(e_hbm.at[pl.ds(0, 1), :],
                                  h_s.at[pl.ds(r, 1), :], sem).wait()
            return carry
        lax.fori_loop(0, R, _drain, 0)

        cx = ctx_ref[:, pl.ds(step, 1), :].reshape(B, D)
        sm = (lax.broadcasted_iota(jnp.int32, (R, B), 0) // K
              == lax.broadcasted_iota(jnp.int32, (R, B), 1))
        hb = jnp.dot(_f32(sm), cx, precision=HI,
                     preferred_element_type=jnp.float32)
        h_s[...] = h_s[...] + hb

    w = w_ref[...]
    h = h_s[...]
    logits = jnp.dot(h, w, preferred_element_type=jnp.float32)
    lanei = lax.broadcasted_iota(jnp.int32, (R, CW), 1)
    lc = jnp.where(lanei + c * CW < V, logits, PADV)

    bms = []
    mb = mb_s[...]
    lane_nbp = lax.broadcasted_iota(jnp.int32, (R, NBP), 1)
    for g in range(NBC):
        lsc[c, g] = lc[:, g * BW:(g + 1) * BW]
        f = jnp.maximum(
            jnp.maximum(lc[:, g * BW:g * BW + 128],
                        lc[:, g * BW + 128:g * BW + 256]),
            jnp.maximum(lc[:, g * BW + 256:g * BW + 384],
                        lc[:, g * BW + 384:g * BW + 512]))
        bm = jnp.max(f, axis=1, keepdims=True)
        mb = jnp.where(lane_nbp == c * NBC + g, bm, mb)
        bms.append(bm)
    mb_s[...] = mb

    @pl.when(c == NC - 1)
    def _merge():
        m = jnp.max(mb_s[...], axis=1, keepdims=True)

        def _sumexp(c2, acc):
            for g in range(NBC):
                e = jnp.exp(lsc[c2, g][...] - m)
                acc = acc + ((e[:, 0:128] + e[:, 128:256])
                             + (e[:, 256:384] + e[:, 384:512]))
            return acc
        svec = lax.fori_loop(0, NC, _sumexp,
                             jnp.zeros((R, 128), jnp.float32))
        logsum = jnp.log(jnp.sum(svec, axis=1, keepdims=True))
        lpr = lpr_s[...]
        ymb = lpr + ((mb_s[...] - m) - logsum)

        lkiota = lax.broadcasted_iota(jnp.int32, (B, K), 1)
        rhs = jnp.concatenate([ymb, lpr, m, logsum], axis=1)
        yks, lpks, mks, lsks = [], [], [], []
        for k in range(K):
            sk = (lax.broadcasted_iota(jnp.int32, (B, R), 1)
                  == K * lax.broadcasted_iota(jnp.int32, (B, R), 0) + k)
            ya = jnp.dot(_f32(sk), rhs, precision=HI,
                         preferred_element_type=jnp.float32)
            yks.append(ya[:, :NBP])
            lpks.append(ya[:, NBP:NBP + 1])
            mks.append(ya[:, NBP + 1:NBP + 2])
            lsks.append(ya[:, NBP + 2:NBP + 3])
        ym4 = jnp.concatenate(yks, axis=1)
        lpk = jnp.concatenate(lpks, axis=1)
        mk = jnp.concatenate(mks, axis=1)
        lsk = jnp.concatenate(lsks, axis=1)

        lane4 = lax.broadcasted_iota(jnp.int32, (B, K * NBP), 1)
        wk = ym4
        sels = []
        for _t in range(8):
            v = jnp.max(wk, axis=1, keepdims=True)
            a = jnp.min(jnp.where(wk == v, lane4, BIGI),
                        axis=1, keepdims=True)
            wk = jnp.where(lane4 == a, PADV, wk)
            sels.append(a)
        asel = jnp.concatenate(sels, axis=1)
        ksel = asel // NBP
        bsel = asel - ksel * NBP
        csel = bsel // NBC
        gsel = bsel - csel * NBC
        biota = lax.broadcasted_iota(jnp.int32, (B, 8), 0)
        srcrow = biota * K + ksel

        stage_s[...] = jnp.concatenate([srcrow, csel, gsel], axis=1)
        cps = pltpu.make_async_copy(stage_s, stage_sm, sem)
        cps.start()
        cps.wait()

        def _gather(b, carry):
            for t in range(8):
                r = stage_sm[b, t]
                cc = stage_sm[b, 8 + t]
                gg = stage_sm[b, 16 + t]
                cand_s[pl.ds(b, 1), t * BW:(t + 1) * BW] = (
                    lsc[cc, gg, pl.ds(r, 1), :])
            return carry
        lax.fori_loop(0, B, _gather, 0)

        iotab = lax.broadcasted_iota(jnp.int32, (B, BW), 1)
        ycand_parts = []
        flat_parts = []
        for t in range(8):
            kt = ksel[:, t:t + 1]
            base = (csel[:, t:t + 1] * CW + gsel[:, t:t + 1] * BW)
            km = lkiota == kt
            lp_t = jnp.sum(jnp.where(km, lpk, 0.0), axis=1, keepdims=True)
            m_t = jnp.sum(jnp.where(km, mk, 0.0), axis=1, keepdims=True)
            ls_t = jnp.sum(jnp.where(km, lsk, 0.0), axis=1, keepdims=True)
            raw = cand_s[:, t * BW:(t + 1) * BW]
            ycand_parts.append(lp_t + ((raw - m_t) - ls_t))
            flat_parts.append(kt * V + base + iotab)
        ycand = jnp.concatenate(ycand_parts, axis=1)
        flatm = jnp.concatenate(flat_parts, axis=1)

        wc = ycand
        lps, flats = [], []
        for _t in range(8):
            v = jnp.max(wc, axis=1, keepdims=True)
            af = jnp.min(jnp.where(wc == v, flatm, BIGI),
                         axis=1, keepdims=True)
            wc = jnp.where(flatm == af, PADV, wc)
            lps.append(v)
            flats.append(af)
        lp8 = jnp.concatenate(lps, axis=1)
        flat8 = jnp.concatenate(flats, axis=1)
        beam8 = flat8 // V
        tok8 = flat8 - beam8 * V

        fs = (tok8 == END).astype(jnp.float32)
        lp2 = lp8 + fs * NEG
        penm = (lax.broadcasted_iota(jnp.int32, (8, L), 1) == step)
        pen_rows = jnp.sum(jnp.where(penm, pen_ref[...], 0.0),
                           axis=1, keepdims=True)
        pen = jnp.max(pen_rows, axis=0, keepdims=True)
        sc8 = lp8 / pen + (1.0 - fs) * NEG

        lane8b = lax.broadcasted_iota(jnp.int32, (B, 8), 1)
        tg_old = tgt_s[...]
        poslane = (lax.broadcasted_iota(jnp.int32, (B, L), 1) == step)

        def topk4_and_rebuild(keys):
            wk2 = keys
            vals, seqs, fsel = [], [], []
            for _j in range(K):
                v = jnp.max(wk2, axis=1, keepdims=True)
                a = jnp.min(jnp.where(wk2 == v, lane8b, BIGI),
                            axis=1, keepdims=True)
                wk2 = jnp.where(lane8b == a, PADV, wk2)
                m8 = lane8b == a
                sbeam = jnp.sum(jnp.where(m8, beam8, 0), axis=1,
                                keepdims=True)
                stok = jnp.sum(jnp.where(m8, tok8, 0), axis=1, keepdims=True)
                sfs = jnp.sum(jnp.where(m8, fs, 0.0), axis=1, keepdims=True)
                seq = jnp.zeros((B, L), jnp.int32)
                for s in range(K):
                    seq = seq + jnp.where(sbeam == s,
                                          tg_old[:, s * L:(s + 1) * L], 0)
                seq = jnp.where(poslane, stok, seq)
                vals.append(v)
                seqs.append(seq)
                fsel.append(sfs)
            return (jnp.concatenate(vals, axis=1),
                    jnp.concatenate(seqs, axis=1),
                    jnp.concatenate(fsel, axis=1))

        logp_new, tgt_new, _ = topk4_and_rebuild(lp2)
        score_new, out_new, flag_new = topk4_and_rebuild(sc8)

        tgt_s[...] = tgt_new
        out_s[...] = out_new
        logp_s[...] = logp_new
        score_s[...] = score_new
        flag_s[...] = flag_new

        smr = (lax.broadcasted_iota(jnp.int32, (R, B), 0) // K
               == lax.broadcasted_iota(jnp.int32, (R, B), 1))
        arep = jnp.dot(_f32(smr), logp_new, precision=HI,
                       preferred_element_type=jnp.float32)
        selc = (lax.broadcasted_iota(jnp.int32, (R, K), 1)
                == lax.broadcasted_iota(jnp.int32, (R, K), 0) % K)
        lpr_s[...] = jnp.sum(jnp.where(selc, arep, 0.0),
                             axis=1, keepdims=True)

        @pl.when(i == NSTEP - 1)
        def _fin():
            done = jnp.max(flag_new, axis=1, keepdims=True)
            out_ref[...] = jnp.where(done > 0, out_new, tgt_new)
            score_ref[...] = jnp.where(done > 0, score_new, logp_new)


def kernel(x, ctx, E, W):
    wp = jnp.pad(W, ((0, 0), (0, VP - V)))
    pens = np.zeros((8, L), np.float32)
    for s in range(L):
        pens[:, s] = np.float32(((5.0 + (s + 1)) / 6.0) ** ALPHA)
    pens_j = jnp.asarray(pens)

    out2, score = pl.pallas_call(
        _body,
        grid=(NSTEP, NC),
        in_specs=[
            pl.BlockSpec((B, L), lambda i, c: (0, 0)),
            pl.BlockSpec((B, L, D), lambda i, c: (0, 0, 0)),
            pl.BlockSpec((8, L), lambda i, c: (0, 0)),
            pl.BlockSpec(memory_space=pltpu.MemorySpace.HBM),
            pl.BlockSpec((D, CW), lambda i, c: (0, c)),
        ],
        out_specs=[
            pl.BlockSpec((B, K * L), lambda i, c: (0, 0)),
            pl.BlockSpec((B, K), lambda i, c: (0, 0)),
        ],
        out_shape=[
            jax.ShapeDtypeStruct((B, K * L), jnp.int32),
            jax.ShapeDtypeStruct((B, K), jnp.float32),
        ],
        scratch_shapes=[
            pltpu.VMEM((B, K * L), jnp.int32),
            pltpu.VMEM((B, K * L), jnp.int32),
            pltpu.VMEM((B, K), jnp.float32),
            pltpu.VMEM((R, 1), jnp.float32),
            pltpu.VMEM((B, K), jnp.float32),
            pltpu.VMEM((B, K), jnp.float32),
            pltpu.VMEM((R, D), jnp.float32),
            pltpu.VMEM((R, NBP), jnp.float32),
            pltpu.VMEM((NC, NBC, R, BW), jnp.float32),
            pltpu.VMEM((B, 8 * BW), jnp.float32),
            pltpu.VMEM((B, 24), jnp.int32),
            pltpu.VMEM((B, K), jnp.int32),
            pltpu.SMEM((B, 24), jnp.int32),
            pltpu.SMEM((B, K), jnp.int32),
            pltpu.SemaphoreType.DMA,
        ],
        compiler_params=pltpu.CompilerParams(
            dimension_semantics=("arbitrary", "arbitrary")),
    )(x, ctx, pens_j, E, wp)
    return out2.reshape(B, K, L), score


# speedup vs baseline: 24.5287x; 1.0043x over previous
import numpy as np
import jax
import jax.numpy as jnp
from jax import lax
from jax.experimental import pallas as pl
from jax.experimental.pallas import tpu as pltpu

B = 32
K = 4
V = 100000
ALPHA = 0.6
END = 2
L = 8
D = 64
NEG = -1e9
PADV = -1e30
R = B * K
CW = 6144
NC = 17
VP = NC * CW
BW = 512
NBC = CW // BW
NB = NC * NBC
NBP = 256
NSTEP = L - 1
BIGI = 2 ** 30
HI = jax.lax.Precision.HIGHEST


def _f32(x):
    return x.astype(jnp.float32)


def _body(x_ref, ctx_ref, pen_ref, e_hbm, w_ref,
          out_ref, score_ref,
          tgt_s, out_s, logp_s, lpr_s, score_s, flag_s,
          h_s, mb_s, lsc, cand_s, stage_s,
          ptok_s, stage_sm, tok_sm, sem):
    i = pl.program_id(0)
    c = pl.program_id(1)
    step = i + 1

    @pl.when(c == 0)
    def _prologue():
        @pl.when(i == 0)
        def _init():
            xv = x_ref[...]
            t0 = jnp.concatenate([xv] * K, axis=1)
            tgt_s[...] = t0
            out_s[...] = t0
            kl = lax.broadcasted_iota(jnp.int32, (B, K), 1)
            logp_s[...] = jnp.where(kl == 0, 0.0, NEG).astype(jnp.float32)
            rl = lax.broadcasted_iota(jnp.int32, (R, 1), 0)
            lpr_s[...] = jnp.where(rl % K == 0, 0.0, NEG).astype(jnp.float32)
            score_s[...] = jnp.full((B, K), NEG, jnp.float32)
            flag_s[...] = jnp.zeros((B, K), jnp.float32)

        mb_s[...] = jnp.full((R, NBP), PADV, jnp.float32)

        lanes = lax.broadcasted_iota(jnp.int32, (B, K * L), 1)
        pmask = (lanes % L) == (step - 1)
        tg = tgt_s[...]
        masked = jnp.where(pmask, tg, 0)
        gm = (lax.broadcasted_iota(jnp.int32, (K * L, K), 0) // L
              == lax.broadcasted_iota(jnp.int32, (K * L, K), 1))
        prev = jnp.dot(_f32(masked), _f32(gm), precision=HI,
                       preferred_element_type=jnp.float32)
        ptok_s[...] = prev.astype(jnp.int32)

        cp = pltpu.make_async_copy(ptok_s, tok_sm, sem)
        cp.start()
        cp.wait()

        def _issue(r, carry):
            b = r // K
            k = r - b * K
            t = tok_sm[b, k]
            pltpu.make_async_copy(e_hbm.at[pl.ds(t, 1), :],
                                  h_s.at[pl.ds(r, 1), :], sem).start()
            return carry
        lax.fori_loop(0, R, _issue, 0)

        pltpu.make_async_copy(e_hbm.at[pl.ds(0, R), :], h_s, sem).wait()

        cx = ctx_ref[:, pl.ds(step, 1), :].reshape(B, D)
        sm = (lax.broadcasted_iota(jnp.int32, (R, B), 0) // K
              == lax.broadcasted_iota(jnp.int32, (R, B), 1))
        hb = jnp.dot(_f32(sm), cx, precision=HI,
                     preferred_element_type=jnp.float32)
        h_s[...] = h_s[...] + hb

    w = w_ref[...]
    h = h_s[...]
    logits = jnp.dot(h, w, preferred_element_type=jnp.float32)
    lanei = lax.broadcasted_iota(jnp.int32, (R, CW), 1)
    lc = jnp.where(lanei + c * CW < V, logits, PADV)

    bms = []
    mb = mb_s[...]
    lane_nbp = lax.broadcasted_iota(jnp.int32, (R, NBP), 1)
    for g in range(NBC):
        lsc[c, g] = lc[:, g * BW:(g + 1) * BW]
        f = jnp.maximum(
            jnp.maximum(lc[:, g * BW:g * BW + 128],
                        lc[:, g * BW + 128:g * BW + 256]),
            jnp.maximum(lc[:, g * BW + 256:g * BW + 384],
                        lc[:, g * BW + 384:g * BW + 512]))
        bm = jnp.max(f, axis=1, keepdims=True)
        mb = jnp.where(lane_nbp == c * NBC + g, bm, mb)
        bms.append(bm)
    mb_s[...] = mb

    @pl.when(c == NC - 1)
    def _merge():
        m = jnp.max(mb_s[...], axis=1, keepdims=True)

        def _sumexp(c2, acc):
            for g in range(NBC):
                e = jnp.exp(lsc[c2, g][...] - m)
                acc = acc + ((e[:, 0:128] + e[:, 128:256])
                             + (e[:, 256:384] + e[:, 384:512]))
            return acc
        svec = lax.fori_loop(0, NC, _sumexp,
                             jnp.zeros((R, 128), jnp.float32))
        logsum = jnp.log(jnp.sum(svec, axis=1, keepdims=True))
        lpr = lpr_s[...]
        ymb = lpr + ((mb_s[...] - m) - logsum)

        lkiota = lax.broadcasted_iota(jnp.int32, (B, K), 1)
        rhs = jnp.concatenate([ymb, lpr, m, logsum], axis=1)
        yks, lpks, mks, lsks = [], [], [], []
        for k in range(K):
            sk = (lax.broadcasted_iota(jnp.int32, (B, R), 1)
                  == K * lax.broadcasted_iota(jnp.int32, (B, R), 0) + k)
            ya = jnp.dot(_f32(sk), rhs, precision=HI,
                         preferred_element_type=jnp.float32)
            yks.append(ya[:, :NBP])
            lpks.append(ya[:, NBP:NBP + 1])
            mks.append(ya[:, NBP + 1:NBP + 2])
            lsks.append(ya[:, NBP + 2:NBP + 3])
        ym4 = jnp.concatenate(yks, axis=1)
        lpk = jnp.concatenate(lpks, axis=1)
        mk = jnp.concatenate(mks, axis=1)
        lsk = jnp.concatenate(lsks, axis=1)

        lane4 = lax.broadcasted_iota(jnp.int32, (B, K * NBP), 1)
        wk = ym4
        sels = []
        for _t in range(8):
            v = jnp.max(wk, axis=1, keepdims=True)
            a = jnp.min(jnp.where(wk == v, lane4, BIGI),
                        axis=1, keepdims=True)
            wk = jnp.where(lane4 == a, PADV, wk)
            sels.append(a)
        asel = jnp.concatenate(sels, axis=1)
        ksel = asel // NBP
        bsel = asel - ksel * NBP
        csel = bsel // NBC
        gsel = bsel - csel * NBC
        biota = lax.broadcasted_iota(jnp.int32, (B, 8), 0)
        srcrow = biota * K + ksel

        stage_s[...] = jnp.concatenate([srcrow, csel, gsel], axis=1)
        cps = pltpu.make_async_copy(stage_s, stage_sm, sem)
        cps.start()
        cps.wait()

        def _gather(b, carry):
            for t in range(8):
                r = stage_sm[b, t]
                cc = stage_sm[b, 8 + t]
                gg = stage_sm[b, 16 + t]
                cand_s[pl.ds(b, 1), t * BW:(t + 1) * BW] = (
                    lsc[cc, gg, pl.ds(r, 1), :])
            return carry
        lax.fori_loop(0, B, _gather, 0)

        iotab = lax.broadcasted_iota(jnp.int32, (B, BW), 1)
        ycand_parts = []
        flat_parts = []
        for t in range(8):
            kt = ksel[:, t:t + 1]
            base = (csel[:, t:t + 1] * CW + gsel[:, t:t + 1] * BW)
            km = lkiota == kt
            lp_t = jnp.sum(jnp.where(km, lpk, 0.0), axis=1, keepdims=True)
            m_t = jnp.sum(jnp.where(km, mk, 0.0), axis=1, keepdims=True)
            ls_t = jnp.sum(jnp.where(km, lsk, 0.0), axis=1, keepdims=True)
            raw = cand_s[:, t * BW:(t + 1) * BW]
            ycand_parts.append(lp_t + ((raw - m_t) - ls_t))
            flat_parts.append(kt * V + base + iotab)
        ycand = jnp.concatenate(ycand_parts, axis=1)
        flatm = jnp.concatenate(flat_parts, axis=1)

        wc = ycand
        lps, flats = [], []
        for _t in range(8):
            v = jnp.max(wc, axis=1, keepdims=True)
            af = jnp.min(jnp.where(wc == v, flatm, BIGI),
                         axis=1, keepdims=True)
            wc = jnp.where(flatm == af, PADV, wc)
            lps.append(v)
            flats.append(af)
        lp8 = jnp.concatenate(lps, axis=1)
        flat8 = jnp.concatenate(flats, axis=1)
        beam8 = flat8 // V
        tok8 = flat8 - beam8 * V

        fs = (tok8 == END).astype(jnp.float32)
        lp2 = lp8 + fs * NEG
        penm = (lax.broadcasted_iota(jnp.int32, (8, L), 1) == step)
        pen_rows = jnp.sum(jnp.where(penm, pen_ref[...], 0.0),
                           axis=1, keepdims=True)
        pen = jnp.max(pen_rows, axis=0, keepdims=True)
        sc8 = lp8 / pen + (1.0 - fs) * NEG

        lane8b = lax.broadcasted_iota(jnp.int32, (B, 8), 1)
        tg_old = tgt_s[...]
        poslane = (lax.broadcasted_iota(jnp.int32, (B, L), 1) == step)

        def topk4_and_rebuild(keys):
            wk2 = keys
            vals, seqs, fsel = [], [], []
            for _j in range(K):
                v = jnp.max(wk2, axis=1, keepdims=True)
                a = jnp.min(jnp.where(wk2 == v, lane8b, BIGI),
                            axis=1, keepdims=True)
                wk2 = jnp.where(lane8b == a, PADV, wk2)
                m8 = lane8b == a
                sbeam = jnp.sum(jnp.where(m8, beam8, 0), axis=1,
                                keepdims=True)
                stok = jnp.sum(jnp.where(m8, tok8, 0), axis=1, keepdims=True)
                sfs = jnp.sum(jnp.where(m8, fs, 0.0), axis=1, keepdims=True)
                seq = jnp.zeros((B, L), jnp.int32)
                for s in range(K):
                    seq = seq + jnp.where(sbeam == s,
                                          tg_old[:, s * L:(s + 1) * L], 0)
                seq = jnp.where(poslane, stok, seq)
                vals.append(v)
                seqs.append(seq)
                fsel.append(sfs)
            return (jnp.concatenate(vals, axis=1),
                    jnp.concatenate(seqs, axis=1),
                    jnp.concatenate(fsel, axis=1))

        logp_new, tgt_new, _ = topk4_and_rebuild(lp2)
        score_new, out_new, flag_new = topk4_and_rebuild(sc8)

        tgt_s[...] = tgt_new
        out_s[...] = out_new
        logp_s[...] = logp_new
        score_s[...] = score_new
        flag_s[...] = flag_new

        smr = (lax.broadcasted_iota(jnp.int32, (R, B), 0) // K
               == lax.broadcasted_iota(jnp.int32, (R, B), 1))
        arep = jnp.dot(_f32(smr), logp_new, precision=HI,
                       preferred_element_type=jnp.float32)
        selc = (lax.broadcasted_iota(jnp.int32, (R, K), 1)
                == lax.broadcasted_iota(jnp.int32, (R, K), 0) % K)
        lpr_s[...] = jnp.sum(jnp.where(selc, arep, 0.0),
                             axis=1, keepdims=True)

        @pl.when(i == NSTEP - 1)
        def _fin():
            done = jnp.max(flag_new, axis=1, keepdims=True)
            out_ref[...] = jnp.where(done > 0, out_new, tgt_new)
            score_ref[...] = jnp.where(done > 0, score_new, logp_new)


def kernel(x, ctx, E, W):
    wp = jnp.pad(W, ((0, 0), (0, VP - V)))
    pens = np.zeros((8, L), np.float32)
    for s in range(L):
        pens[:, s] = np.float32(((5.0 + (s + 1)) / 6.0) ** ALPHA)
    pens_j = jnp.asarray(pens)

    out2, score = pl.pallas_call(
        _body,
        grid=(NSTEP, NC),
        in_specs=[
            pl.BlockSpec((B, L), lambda i, c: (0, 0)),
            pl.BlockSpec((B, L, D), lambda i, c: (0, 0, 0)),
            pl.BlockSpec((8, L), lambda i, c: (0, 0)),
            pl.BlockSpec(memory_space=pltpu.MemorySpace.HBM),
            pl.BlockSpec((D, CW), lambda i, c: (0, c)),
        ],
        out_specs=[
            pl.BlockSpec((B, K * L), lambda i, c: (0, 0)),
            pl.BlockSpec((B, K), lambda i, c: (0, 0)),
        ],
        out_shape=[
            jax.ShapeDtypeStruct((B, K * L), jnp.int32),
            jax.ShapeDtypeStruct((B, K), jnp.float32),
        ],
        scratch_shapes=[
            pltpu.VMEM((B, K * L), jnp.int32),
            pltpu.VMEM((B, K * L), jnp.int32),
            pltpu.VMEM((B, K), jnp.float32),
            pltpu.VMEM((R, 1), jnp.float32),
            pltpu.VMEM((B, K), jnp.float32),
            pltpu.VMEM((B, K), jnp.float32),
            pltpu.VMEM((R, D), jnp.float32),
            pltpu.VMEM((R, NBP), jnp.float32),
            pltpu.VMEM((NC, NBC, R, BW), jnp.float32),
            pltpu.VMEM((B, 8 * BW), jnp.float32),
            pltpu.VMEM((B, 24), jnp.int32),
            pltpu.VMEM((B, K), jnp.int32),
            pltpu.SMEM((B, 24), jnp.int32),
            pltpu.SMEM((B, K), jnp.int32),
            pltpu.SemaphoreType.DMA,
        ],
        compiler_params=pltpu.CompilerParams(
            dimension_semantics=("arbitrary", "arbitrary")),
    )(x, ctx, pens_j, E, wp)
    return out2.reshape(B, K, L), score
